# Initial kernel scaffold; baseline (speedup 1.0000x reference)
import jax
import jax.numpy as jnp
from jax.experimental import pallas as pl


def kernel(t, v, edge_index, conductance, theta_sd_1, theta_sd_2):
    raise NotImplementedError("write your pallas kernel here")


# speedup vs baseline: 100.7112x; 100.7112x over previous
import functools

import jax
import jax.numpy as jnp
from jax import lax
from jax.experimental import pallas as pl
from jax.experimental.pallas import tpu as pltpu
from jax.experimental.pallas import tpu_sc as plsc

N_NODES = 50000
N_EDGES = 1600000
THETA_CONST = 1.0

L = 16
NC, NS = 2, 16
NW = NC * NS
E_PER_W = N_EDGES // NW
CHUNK = 2000
NCHUNK = E_PER_W // CHUNK
STEPS = CHUNK // L
N_PAD = 50176
R = N_PAD // NS


def _edge_flow_body(v_hbm, src_hbm, dst_hbm, g_hbm, t1_hbm, t2_hbm,
                    out_hbm,
                    v_vm, acc, sbuf, dbuf, gbuf, t1buf, t2buf,
                    shared, tmp, red):
    cid = lax.axis_index("c")
    sid = lax.axis_index("s")
    wid = cid * NS + sid
    ebase = wid * E_PER_W

    pltpu.sync_copy(v_hbm, v_vm)

    zero = jnp.zeros((L,), jnp.float32)

    def zbody(i, c):
        acc[pl.ds(i * L, L)] = zero
        return c

    lax.fori_loop(0, N_PAD // L, zbody, 0)

    def chunk_body(gi, c):
        base = ebase + gi * CHUNK
        pltpu.sync_copy(src_hbm.at[pl.ds(base, CHUNK)], sbuf)
        pltpu.sync_copy(dst_hbm.at[pl.ds(base, CHUNK)], dbuf)
        pltpu.sync_copy(g_hbm.at[pl.ds(base, CHUNK)], gbuf)
        pltpu.sync_copy(t1_hbm.at[pl.ds(base, CHUNK)], t1buf)
        pltpu.sync_copy(t2_hbm.at[pl.ds(base, CHUNK)], t2buf)

        def step(j, cc):
            off = j * L
            s = sbuf[pl.ds(off, L)]
            d = dbuf[pl.ds(off, L)]
            vs = plsc.load_gather(v_vm, [s])
            vd = plsc.load_gather(v_vm, [d])
            flow = gbuf[pl.ds(off, L)] * jnp.maximum(
                t1buf[pl.ds(off, L)] * (vs - vd) + t2buf[pl.ds(off, L)], 0.0)
            plsc.addupdate_scatter(acc, [d], flow)
            plsc.addupdate_scatter(acc, [s], -flow)
            return cc

        lax.fori_loop(0, STEPS, step, 0)
        return c

    lax.fori_loop(0, NCHUNK, chunk_body, 0)

    rbase = sid * R

    def cbody(k, c):
        o = k * L
        red[pl.ds(o, L)] = acc[pl.ds(rbase + o, L)]
        return c

    lax.fori_loop(0, R // L, cbody, 0)

    for r in range(1, NS):
        dpub = lax.rem(sid + r, NS)
        jsrc = lax.rem(sid + NS - r, NS)
        pltpu.sync_copy(acc.at[pl.ds(pl.multiple_of(dpub * R, 8), R)],
                        shared.at[pl.ds(sid * R, R)])
        plsc.subcore_barrier()
        pltpu.sync_copy(shared.at[pl.ds(pl.multiple_of(jsrc * R, 8), R)],
                        tmp)

        def abody(k, c):
            o = k * L
            red[pl.ds(o, L)] = red[pl.ds(o, L)] + tmp[pl.ds(o, L)]
            return c

        lax.fori_loop(0, R // L, abody, 0)
        plsc.subcore_barrier()

    pltpu.sync_copy(red, out_hbm.at[pl.ds(cid * N_PAD + rbase, R)])


_edge_flow = functools.partial(
    pl.kernel,
    out_type=jax.ShapeDtypeStruct((NC * N_PAD,), jnp.float32),
    mesh=plsc.VectorSubcoreMesh(core_axis_name="c", subcore_axis_name="s"),
    compiler_params=pltpu.CompilerParams(needs_layout_passes=False),
    scratch_types=[
        pltpu.VMEM((N_NODES,), jnp.float32),
        pltpu.VMEM((N_PAD,), jnp.float32),
        pltpu.VMEM((CHUNK,), jnp.int32),
        pltpu.VMEM((CHUNK,), jnp.int32),
        pltpu.VMEM((CHUNK,), jnp.float32),
        pltpu.VMEM((CHUNK,), jnp.float32),
        pltpu.VMEM((CHUNK,), jnp.float32),
        pltpu.VMEM_SHARED((N_PAD,), jnp.float32),
        pltpu.VMEM((R,), jnp.float32),
        pltpu.VMEM((R,), jnp.float32),
    ],
)(_edge_flow_body)


def _combine_body(p_ref, o_ref):
    o_ref[...] = (p_ref[0] + p_ref[1]) * (1.0 / THETA_CONST)


def _combine(partials):
    out = pl.pallas_call(
        _combine_body,
        out_shape=jax.ShapeDtypeStruct((N_PAD // 128, 128), jnp.float32),
    )(partials.reshape(NC, N_PAD // 128, 128))
    return out.reshape(N_PAD)[:N_NODES]


def kernel(t, v, edge_index, conductance, theta_sd_1, theta_sd_2):
    del t
    src = edge_index[0]
    dst = edge_index[1]
    partials = _edge_flow(v, src, dst, conductance, theta_sd_1, theta_sd_2)
    return _combine(partials.reshape(NC, N_PAD))
